# baseline (speedup 1.0000x reference)
import functools

import jax
import jax.numpy as jnp
from jax import lax
from jax.experimental import pallas as pl
from jax.experimental.pallas import tpu as pltpu
from jax.experimental.pallas import tpu_sc as plsc

_N = 10000
_NP = 10240
_E = 320000
_D = 128
_B = 256

_NC, _NS = 2, 16
_NW = _NC * _NS
_EPT = _E // _NW
_CH = 128
_NCH = 80
_QC = _NCH // 4
_QE = _QC * _CH
_EPTP = _NCH * _CH
_RPT = _NP // _NS

_BLK = 1000


def _proj_body(x_ref, w_ref, asrc_ref, adst_ref, h_ref, es_ref, ed_ref):
    h = jnp.dot(x_ref[...], w_ref[...], preferred_element_type=jnp.float32)
    h_ref[...] = h
    es_ref[...] = jnp.sum(h * asrc_ref[...], axis=1, keepdims=True)
    ed_ref[...] = jnp.sum(h * adst_ref[...], axis=1, keepdims=True)


def _proj(x, W, a_src, a_dst):
    return pl.pallas_call(
        _proj_body,
        grid=(_N // _BLK,),
        in_specs=[
            pl.BlockSpec((_BLK, _D), lambda i: (i, 0)),
            pl.BlockSpec((_D, _D), lambda i: (0, 0)),
            pl.BlockSpec((1, _D), lambda i: (0, 0)),
            pl.BlockSpec((1, _D), lambda i: (0, 0)),
        ],
        out_specs=[
            pl.BlockSpec((_BLK, _D), lambda i: (i, 0)),
            pl.BlockSpec((_BLK, 1), lambda i: (i, 0)),
            pl.BlockSpec((_BLK, 1), lambda i: (i, 0)),
        ],
        out_shape=[
            jax.ShapeDtypeStruct((_N, _D), jnp.float32),
            jax.ShapeDtypeStruct((_N, 1), jnp.float32),
            jax.ShapeDtypeStruct((_N, 1), jnp.float32),
        ],
    )(x, W, a_src.reshape(1, _D), a_dst.reshape(1, _D))


def _merge_proj_body(p0_ref, p1_ref, dp_ref, b_ref, w_ref, asrc_ref, adst_ref,
                     h_ref, es_ref, ed_ref):
    den = jnp.sum(dp_ref[...], axis=1, keepdims=True)
    z = (p0_ref[...] + p1_ref[...]) / (den + 1e-16) + b_ref[...]
    act = jnp.where(z > 0, z, 0.01 * z)
    h = jnp.dot(act, w_ref[...], preferred_element_type=jnp.float32)
    h_ref[...] = h
    es_ref[...] = jnp.sum(h * asrc_ref[...], axis=1, keepdims=True)
    ed_ref[...] = jnp.sum(h * adst_ref[...], axis=1, keepdims=True)


def _merge_proj(p0, p1, dpT, b, W, a_src, a_dst):
    return pl.pallas_call(
        _merge_proj_body,
        grid=(_N // _BLK,),
        in_specs=[
            pl.BlockSpec((_BLK, _D), lambda i: (i, 0)),
            pl.BlockSpec((_BLK, _D), lambda i: (i, 0)),
            pl.BlockSpec((_BLK, _NW), lambda i: (i, 0)),
            pl.BlockSpec((1, _D), lambda i: (0, 0)),
            pl.BlockSpec((_D, _D), lambda i: (0, 0)),
            pl.BlockSpec((1, _D), lambda i: (0, 0)),
            pl.BlockSpec((1, _D), lambda i: (0, 0)),
        ],
        out_specs=[
            pl.BlockSpec((_BLK, _D), lambda i: (i, 0)),
            pl.BlockSpec((_BLK, 1), lambda i: (i, 0)),
            pl.BlockSpec((_BLK, 1), lambda i: (i, 0)),
        ],
        out_shape=[
            jax.ShapeDtypeStruct((_N, _D), jnp.float32),
            jax.ShapeDtypeStruct((_N, 1), jnp.float32),
            jax.ShapeDtypeStruct((_N, 1), jnp.float32),
        ],
    )(p0, p1, dpT, b.reshape(1, _D), W, a_src.reshape(1, _D),
      a_dst.reshape(1, _D))


def _final_body(p0_ref, p1_ref, dp_ref, b_ref, batch_ref, fcw1_ref, fcb1_ref,
                gamma_ref, beta_ref, fcw2_ref, fcb2_ref, out_ref,
                pool_acc, cnt_acc):
    i = pl.program_id(0)

    @pl.when(i == 0)
    def _():
        pool_acc[...] = jnp.zeros_like(pool_acc)
        cnt_acc[...] = jnp.zeros_like(cnt_acc)

    den = jnp.sum(dp_ref[...], axis=1, keepdims=True)
    z = (p0_ref[...] + p1_ref[...]) / (den + 1e-16) + b_ref[...]
    act = jnp.where(z > 0, z, 0.01 * z)
    bids = batch_ref[...].reshape(1, _BLK)
    mask = (bids == lax.broadcasted_iota(jnp.int32, (_B, _BLK), 0))
    mask = mask.astype(jnp.float32)
    pool_acc[...] += jnp.dot(mask, act, preferred_element_type=jnp.float32,
                             precision=lax.Precision.HIGHEST)
    cnt_acc[...] += jnp.sum(mask, axis=1, keepdims=True)

    @pl.when(i == pl.num_programs(0) - 1)
    def _():
        pooled = pool_acc[...] / jnp.maximum(cnt_acc[...], 1.0)
        z1 = jnp.dot(pooled, fcw1_ref[...], preferred_element_type=jnp.float32)
        z1 = z1 + fcb1_ref[...]
        mu = jnp.mean(z1, axis=0, keepdims=True)
        var = jnp.mean((z1 - mu) * (z1 - mu), axis=0, keepdims=True)
        z1 = (z1 - mu) / jnp.sqrt(var + 1e-5) * gamma_ref[...] + beta_ref[...]
        z1 = jnp.where(z1 > 0, z1, 0.01 * z1)
        z2 = jnp.dot(z1, fcw2_ref[...], preferred_element_type=jnp.float32)
        out_ref[...] = z2 + fcb2_ref[...]


def _final(p0, p1, dpT, b, batch3, fcW1, fcb1, gamma, beta, fcW2, fcb2):
    return pl.pallas_call(
        _final_body,
        grid=(_N // _BLK,),
        in_specs=[
            pl.BlockSpec((_BLK, _D), lambda i: (i, 0)),
            pl.BlockSpec((_BLK, _D), lambda i: (i, 0)),
            pl.BlockSpec((_BLK, _NW), lambda i: (i, 0)),
            pl.BlockSpec((1, _D), lambda i: (0, 0)),
            pl.BlockSpec((1, 1, _BLK), lambda i: (i, 0, 0)),
            pl.BlockSpec((_D, _D), lambda i: (0, 0)),
            pl.BlockSpec((1, _D), lambda i: (0, 0)),
            pl.BlockSpec((1, _D), lambda i: (0, 0)),
            pl.BlockSpec((1, _D), lambda i: (0, 0)),
            pl.BlockSpec((_D, _D), lambda i: (0, 0)),
            pl.BlockSpec((1, _D), lambda i: (0, 0)),
        ],
        out_specs=pl.BlockSpec((_B, _D), lambda i: (0, 0)),
        out_shape=jax.ShapeDtypeStruct((_B, _D), jnp.float32),
        scratch_shapes=[
            pltpu.VMEM((_B, _D), jnp.float32),
            pltpu.VMEM((_B, 1), jnp.float32),
        ],
    )(p0, p1, dpT, b.reshape(1, _D), batch3, fcW1, fcb1.reshape(1, _D),
      gamma.reshape(1, _D), beta.reshape(1, _D), fcW2, fcb2.reshape(1, _D))


def _edge_w_body(srcp_hbm, dstp_hbm, es_hbm, ed_hbm, ex_hbm, den_hbm,
                 srcv, dstv, esv, edv, exv, denv):
    c = lax.axis_index("c")
    s = lax.axis_index("s")
    wid = c * _NS + s

    pltpu.sync_copy(es_hbm, esv)
    pltpu.sync_copy(ed_hbm, edv)
    pltpu.sync_copy(srcp_hbm.at[pl.ds(wid * _EPTP, _EPTP)], srcv)
    pltpu.sync_copy(dstp_hbm.at[pl.ds(wid * _EPTP, _EPTP)], dstv)

    zeros16 = jnp.zeros((16,), jnp.float32)

    def _zden(i, carry):
        denv[pl.ds(i * 16, 16)] = zeros16
        return carry

    lax.fori_loop(0, _N // 16, _zden, 0)

    lane = lax.iota(jnp.int32, 16)

    def _edge(i, carry):
        sl = pl.ds(i * 16, 16)
        sidx = srcv[sl]
        didx = dstv[sl]
        e = plsc.load_gather(esv, [sidx]) + plsc.load_gather(edv, [didx])
        e = jnp.where(e > 0, e, 0.2 * e)
        ex = jnp.exp(e)
        ex = jnp.where(i * 16 + lane < _EPT, ex, 0.0)
        exv[sl] = ex
        for l in range(16):
            plsc.addupdate_scatter(denv, [didx], ex, mask=lane == l)
        return carry

    lax.fori_loop(0, _EPTP // 16, _edge, 0)
    pltpu.sync_copy(exv, ex_hbm.at[pl.ds(wid * _EPTP, _EPTP)])
    pltpu.sync_copy(denv, den_hbm.at[pl.ds(wid * _N, _N)])


def _edge_w(srcp, dstp, es, ed):
    mesh = plsc.VectorSubcoreMesh(core_axis_name="c", subcore_axis_name="s")
    f = pl.kernel(
        _edge_w_body,
        out_type=[
            jax.ShapeDtypeStruct((_NW * _EPTP,), jnp.float32),
            jax.ShapeDtypeStruct((_NW * _N,), jnp.float32),
        ],
        mesh=mesh,
        compiler_params=pltpu.CompilerParams(needs_layout_passes=False),
        scratch_types=[
            pltpu.VMEM((_EPTP,), jnp.int32),
            pltpu.VMEM((_EPTP,), jnp.int32),
            pltpu.VMEM((_N,), jnp.float32),
            pltpu.VMEM((_N,), jnp.float32),
            pltpu.VMEM((_EPTP,), jnp.float32),
            pltpu.VMEM((_N,), jnp.float32),
        ],
    )
    return f(srcp, dstp, es, ed)


_NB = 2


def _agg_body(h_hbm, s2d_hbm, d2d_hbm, ex2d_hbm, out_hbm,
              s2d, d2d, ex2d, rb0, rb1, out_sh, gs0, gs1):
    c = lax.axis_index("c")
    s = lax.axis_index("s")
    wid = c * _NS + s
    rbase = s * _RPT
    rbs = (rb0, rb1)
    gss = (gs0, gs1)

    pltpu.sync_copy(s2d_hbm.at[pl.ds(wid * _EPTP, _EPTP)], s2d)
    pltpu.sync_copy(d2d_hbm.at[pl.ds(wid * _EPTP, _QE)], d2d)
    pltpu.sync_copy(ex2d_hbm.at[pl.ds(wid * _EPTP, _QE)], ex2d)

    zeros16 = jnp.zeros((16,), jnp.float32)

    def _zrow(r, carry):
        for j in range(_D // 16):
            rb0[r, pl.ds(j * 16, 16)] = zeros16
        return carry

    lax.fori_loop(0, _CH, _zrow, 0)

    for j in range(_RPT // _CH):
        pltpu.sync_copy(rb0, out_sh.at[pl.ds(rbase + j * _CH, _CH)])
    plsc.subcore_barrier()

    for b in range(_NB - 1):
        pltpu.async_copy(h_hbm.at[s2d.at[pl.ds(b * _CH, _CH)]],
                         rbs[b], gss[b])

    def _outer(it, carry):
        base = it * _NB
        for b in range(_NB):
            ci = base + b
            pb = (b + _NB - 1) % _NB
            qi = lax.rem(ci, _QC)

            @pl.when(jnp.logical_and(qi == 0, ci > 0))
            def _(ci=ci):
                off = wid * _EPTP + (ci // _QC) * _QE
                pltpu.sync_copy(d2d_hbm.at[pl.ds(off, _QE)], d2d)
                pltpu.sync_copy(ex2d_hbm.at[pl.ds(off, _QE)], ex2d)

            cn = jnp.minimum(ci + _NB - 1, _NCH - 1)
            pltpu.async_copy(h_hbm.at[s2d.at[pl.ds(cn * _CH, _CH)]],
                             rbs[pb], gss[pb])

            pltpu.make_async_copy(h_hbm.at[s2d.at[pl.ds(ci * _CH, _CH)]],
                                  rbs[b], gss[b]).wait()
            rb = rbs[b]
            for g in range(_CH // 16):
                exg = ex2d[pl.ds(qi * _CH + g * 16, 16)]
                for l in range(16):
                    r = g * 16 + l
                    av = lax.broadcast(exg[l], (16,))
                    for j in range(_D // 16):
                        sl = pl.ds(j * 16, 16)
                        rb[r, sl] = rb[r, sl] * av
            pltpu.sync_copy(rb, out_sh.at[d2d.at[pl.ds(qi * _CH, _CH)]],
                            add=True)
        return carry

    lax.fori_loop(0, _NCH // _NB, _outer, 0)
    pltpu.make_async_copy(
        h_hbm.at[s2d.at[pl.ds((_NCH - 1) * _CH, _CH)]],
        rbs[_NCH % _NB], gss[_NCH % _NB]).wait()

    plsc.subcore_barrier()
    pltpu.sync_copy(out_sh.at[pl.ds(rbase, _RPT)],
                    out_hbm.at[c, pl.ds(rbase, _RPT)])


def _agg(h, s2d, d2d, ex2d):
    mesh = plsc.VectorSubcoreMesh(core_axis_name="c", subcore_axis_name="s")
    f = pl.kernel(
        _agg_body,
        out_type=jax.ShapeDtypeStruct((_NC, _NP, _D), jnp.float32),
        mesh=mesh,
        compiler_params=pltpu.CompilerParams(needs_layout_passes=False),
        scratch_types=[
            pltpu.VMEM((_EPTP,), jnp.int32),
            pltpu.VMEM((_QE,), jnp.int32),
            pltpu.VMEM((_QE,), jnp.float32),
            pltpu.VMEM((_CH, _D), jnp.float32),
            pltpu.VMEM((_CH, _D), jnp.float32),
            pltpu.VMEM_SHARED((_NP, _D), jnp.float32),
            pltpu.SemaphoreType.DMA,
            pltpu.SemaphoreType.DMA,
        ],
    )
    return f(h, s2d, d2d, ex2d)


def _gat_sc(h, srcp, dstp, es, ed):
    ex, den_p = _edge_w(srcp, dstp, es, ed)
    out_p = _agg(h, srcp, dstp, ex)
    return out_p, den_p.reshape(_NW, _N)


@jax.jit
def kernel(x, edge_index, batch, W1, a_src1, a_dst1, b1, W2, a_src2, a_dst2,
           b2, fcW1, fcb1, gamma, beta, fcW2, fcb2):
    srcp = jnp.pad(edge_index[0].reshape(_NW, _EPT),
                   ((0, 0), (0, _EPTP - _EPT))).reshape(_NW * _EPTP)
    dstp = jnp.pad(edge_index[1].reshape(_NW, _EPT),
                   ((0, 0), (0, _EPTP - _EPT))).reshape(_NW * _EPTP)
    batch3 = batch.reshape(_N // _BLK, 1, _BLK)

    h1, es1, ed1 = _proj(x, W1, a_src1, a_dst1)
    out_p1, den_p1 = _gat_sc(h1, srcp, dstp,
                             es1.reshape(_N), ed1.reshape(_N))
    h2, es2, ed2 = _merge_proj(out_p1[0, :_N], out_p1[1, :_N], den_p1.T, b1,
                               W2, a_src2, a_dst2)
    out_p2, den_p2 = _gat_sc(h2, srcp, dstp,
                             es2.reshape(_N), ed2.reshape(_N))
    return _final(out_p2[0, :_N], out_p2[1, :_N], den_p2.T, b2, batch3,
                  fcW1, fcb1, gamma, beta, fcW2, fcb2)


# speedup vs baseline: 1.1874x; 1.1874x over previous
import functools

import jax
import jax.numpy as jnp
from jax import lax
from jax.experimental import pallas as pl
from jax.experimental.pallas import tpu as pltpu
from jax.experimental.pallas import tpu_sc as plsc

_N = 10000
_NP = 10240
_E = 320000
_D = 128
_B = 256

_NC, _NS = 2, 16
_NW = _NC * _NS
_EPT = _E // _NW
_CH = 128
_NCH = 80
_EPTP = _NCH * _CH
_RPT = _NP // _NS

_BLK = 1000


def _proj_body(x_ref, w_ref, asrc_ref, adst_ref, h_ref, es_ref, ed_ref):
    h = jnp.dot(x_ref[...], w_ref[...], preferred_element_type=jnp.float32)
    h_ref[...] = h
    es_ref[...] = jnp.sum(h * asrc_ref[...], axis=1, keepdims=True)
    ed_ref[...] = jnp.sum(h * adst_ref[...], axis=1, keepdims=True)


def _proj(x, W, a_src, a_dst):
    return pl.pallas_call(
        _proj_body,
        grid=(_N // _BLK,),
        in_specs=[
            pl.BlockSpec((_BLK, _D), lambda i: (i, 0)),
            pl.BlockSpec((_D, _D), lambda i: (0, 0)),
            pl.BlockSpec((1, _D), lambda i: (0, 0)),
            pl.BlockSpec((1, _D), lambda i: (0, 0)),
        ],
        out_specs=[
            pl.BlockSpec((_BLK, _D), lambda i: (i, 0)),
            pl.BlockSpec((_BLK, 1), lambda i: (i, 0)),
            pl.BlockSpec((_BLK, 1), lambda i: (i, 0)),
        ],
        out_shape=[
            jax.ShapeDtypeStruct((_N, _D), jnp.float32),
            jax.ShapeDtypeStruct((_N, 1), jnp.float32),
            jax.ShapeDtypeStruct((_N, 1), jnp.float32),
        ],
    )(x, W, a_src.reshape(1, _D), a_dst.reshape(1, _D))


def _merge_proj_body(p0_ref, p1_ref, dp_ref, b_ref, w_ref, asrc_ref, adst_ref,
                     h_ref, es_ref, ed_ref):
    den = jnp.sum(dp_ref[...], axis=1, keepdims=True)
    z = (p0_ref[...] + p1_ref[...]) / (den + 1e-16) + b_ref[...]
    act = jnp.where(z > 0, z, 0.01 * z)
    h = jnp.dot(act, w_ref[...], preferred_element_type=jnp.float32)
    h_ref[...] = h
    es_ref[...] = jnp.sum(h * asrc_ref[...], axis=1, keepdims=True)
    ed_ref[...] = jnp.sum(h * adst_ref[...], axis=1, keepdims=True)


def _merge_proj(p0, p1, dpT, b, W, a_src, a_dst):
    return pl.pallas_call(
        _merge_proj_body,
        grid=(_N // _BLK,),
        in_specs=[
            pl.BlockSpec((_BLK, _D), lambda i: (i, 0)),
            pl.BlockSpec((_BLK, _D), lambda i: (i, 0)),
            pl.BlockSpec((_BLK, _NW), lambda i: (i, 0)),
            pl.BlockSpec((1, _D), lambda i: (0, 0)),
            pl.BlockSpec((_D, _D), lambda i: (0, 0)),
            pl.BlockSpec((1, _D), lambda i: (0, 0)),
            pl.BlockSpec((1, _D), lambda i: (0, 0)),
        ],
        out_specs=[
            pl.BlockSpec((_BLK, _D), lambda i: (i, 0)),
            pl.BlockSpec((_BLK, 1), lambda i: (i, 0)),
            pl.BlockSpec((_BLK, 1), lambda i: (i, 0)),
        ],
        out_shape=[
            jax.ShapeDtypeStruct((_N, _D), jnp.float32),
            jax.ShapeDtypeStruct((_N, 1), jnp.float32),
            jax.ShapeDtypeStruct((_N, 1), jnp.float32),
        ],
    )(p0, p1, dpT, b.reshape(1, _D), W, a_src.reshape(1, _D),
      a_dst.reshape(1, _D))


def _final_body(p0_ref, p1_ref, dp_ref, b_ref, batch_ref, fcw1_ref, fcb1_ref,
                gamma_ref, beta_ref, fcw2_ref, fcb2_ref, out_ref,
                pool_acc, cnt_acc):
    i = pl.program_id(0)

    @pl.when(i == 0)
    def _():
        pool_acc[...] = jnp.zeros_like(pool_acc)
        cnt_acc[...] = jnp.zeros_like(cnt_acc)

    den = jnp.sum(dp_ref[...], axis=1, keepdims=True)
    z = (p0_ref[...] + p1_ref[...]) / (den + 1e-16) + b_ref[...]
    act = jnp.where(z > 0, z, 0.01 * z)
    bids = batch_ref[...].reshape(1, _BLK)
    mask = (bids == lax.broadcasted_iota(jnp.int32, (_B, _BLK), 0))
    mask = mask.astype(jnp.float32)
    pool_acc[...] += jnp.dot(mask, act, preferred_element_type=jnp.float32,
                             precision=lax.Precision.HIGHEST)
    cnt_acc[...] += jnp.sum(mask, axis=1, keepdims=True)

    @pl.when(i == pl.num_programs(0) - 1)
    def _():
        pooled = pool_acc[...] / jnp.maximum(cnt_acc[...], 1.0)
        z1 = jnp.dot(pooled, fcw1_ref[...], preferred_element_type=jnp.float32)
        z1 = z1 + fcb1_ref[...]
        mu = jnp.mean(z1, axis=0, keepdims=True)
        var = jnp.mean((z1 - mu) * (z1 - mu), axis=0, keepdims=True)
        z1 = (z1 - mu) / jnp.sqrt(var + 1e-5) * gamma_ref[...] + beta_ref[...]
        z1 = jnp.where(z1 > 0, z1, 0.01 * z1)
        z2 = jnp.dot(z1, fcw2_ref[...], preferred_element_type=jnp.float32)
        out_ref[...] = z2 + fcb2_ref[...]


def _final(p0, p1, dpT, b, batch3, fcW1, fcb1, gamma, beta, fcW2, fcb2):
    return pl.pallas_call(
        _final_body,
        grid=(_N // _BLK,),
        in_specs=[
            pl.BlockSpec((_BLK, _D), lambda i: (i, 0)),
            pl.BlockSpec((_BLK, _D), lambda i: (i, 0)),
            pl.BlockSpec((_BLK, _NW), lambda i: (i, 0)),
            pl.BlockSpec((1, _D), lambda i: (0, 0)),
            pl.BlockSpec((1, 1, _BLK), lambda i: (i, 0, 0)),
            pl.BlockSpec((_D, _D), lambda i: (0, 0)),
            pl.BlockSpec((1, _D), lambda i: (0, 0)),
            pl.BlockSpec((1, _D), lambda i: (0, 0)),
            pl.BlockSpec((1, _D), lambda i: (0, 0)),
            pl.BlockSpec((_D, _D), lambda i: (0, 0)),
            pl.BlockSpec((1, _D), lambda i: (0, 0)),
        ],
        out_specs=pl.BlockSpec((_B, _D), lambda i: (0, 0)),
        out_shape=jax.ShapeDtypeStruct((_B, _D), jnp.float32),
        scratch_shapes=[
            pltpu.VMEM((_B, _D), jnp.float32),
            pltpu.VMEM((_B, 1), jnp.float32),
        ],
    )(p0, p1, dpT, b.reshape(1, _D), batch3, fcW1, fcb1.reshape(1, _D),
      gamma.reshape(1, _D), beta.reshape(1, _D), fcW2, fcb2.reshape(1, _D))


def _edge_w_body(srcp_hbm, dstp_hbm, es_hbm, ed_hbm, ex_hbm, den_hbm,
                 srcv, dstv, esv, edv, exv, denv):
    c = lax.axis_index("c")
    s = lax.axis_index("s")
    wid = c * _NS + s

    pltpu.sync_copy(es_hbm, esv)
    pltpu.sync_copy(ed_hbm, edv)
    pltpu.sync_copy(srcp_hbm.at[pl.ds(wid * _EPTP, _EPTP)], srcv)
    pltpu.sync_copy(dstp_hbm.at[pl.ds(wid * _EPTP, _EPTP)], dstv)

    zeros16 = jnp.zeros((16,), jnp.float32)

    def _zden(i, carry):
        denv[pl.ds(i * 16, 16)] = zeros16
        return carry

    lax.fori_loop(0, _N // 16, _zden, 0)

    lane = lax.iota(jnp.int32, 16)

    def _edge(i, carry):
        sl = pl.ds(i * 16, 16)
        sidx = srcv[sl]
        didx = dstv[sl]
        e = plsc.load_gather(esv, [sidx]) + plsc.load_gather(edv, [didx])
        e = jnp.where(e > 0, e, 0.2 * e)
        ex = jnp.exp(e)
        ex = jnp.where(i * 16 + lane < _EPT, ex, 0.0)
        exv[sl] = ex
        for l in range(16):
            plsc.addupdate_scatter(denv, [didx], ex, mask=lane == l)
        return carry

    lax.fori_loop(0, _EPTP // 16, _edge, 0)
    pltpu.sync_copy(exv, ex_hbm.at[pl.ds(wid * _EPTP, _EPTP)])
    pltpu.sync_copy(denv, den_hbm.at[pl.ds(wid * _N, _N)])


def _edge_w(srcp, dstp, es, ed):
    mesh = plsc.VectorSubcoreMesh(core_axis_name="c", subcore_axis_name="s")
    f = pl.kernel(
        _edge_w_body,
        out_type=[
            jax.ShapeDtypeStruct((_NW * _EPTP,), jnp.float32),
            jax.ShapeDtypeStruct((_NW * _N,), jnp.float32),
        ],
        mesh=mesh,
        compiler_params=pltpu.CompilerParams(needs_layout_passes=False),
        scratch_types=[
            pltpu.VMEM((_EPTP,), jnp.int32),
            pltpu.VMEM((_EPTP,), jnp.int32),
            pltpu.VMEM((_N,), jnp.float32),
            pltpu.VMEM((_N,), jnp.float32),
            pltpu.VMEM((_EPTP,), jnp.float32),
            pltpu.VMEM((_N,), jnp.float32),
        ],
    )
    return f(srcp, dstp, es, ed)


def _agg_body(h_hbm, s2d_hbm, d2d_hbm, ex2d_hbm, out_hbm,
              s2d, d2d, ex2d, rowbuf, out_sh, sem):
    c = lax.axis_index("c")
    s = lax.axis_index("s")
    wid = c * _NS + s
    rbase = s * _RPT

    pltpu.sync_copy(s2d_hbm.at[pl.ds(wid * _NCH, _NCH)], s2d)
    pltpu.sync_copy(d2d_hbm.at[pl.ds(wid * _NCH, _NCH)], d2d)
    pltpu.sync_copy(ex2d_hbm.at[pl.ds(wid * _NCH, _NCH)], ex2d)

    zeros16 = jnp.zeros((16,), jnp.float32)

    def _zrow(r, carry):
        for j in range(_D // 16):
            rowbuf[r, pl.ds(j * 16, 16)] = zeros16
        return carry

    lax.fori_loop(0, _CH, _zrow, 0)

    for j in range(_RPT // _CH):
        pltpu.sync_copy(rowbuf, out_sh.at[pl.ds(rbase + j * _CH, _CH)])
    plsc.subcore_barrier()

    def _chunk(ci, carry):
        pltpu.async_copy(h_hbm.at[s2d.at[ci]], rowbuf, sem).wait()
        for g in range(_CH // 16):
            exg = ex2d[ci, pl.ds(g * 16, 16)]
            for l in range(16):
                r = g * 16 + l
                av = lax.broadcast(exg[l], (16,))
                for j in range(_D // 16):
                    sl = pl.ds(j * 16, 16)
                    rowbuf[r, sl] = rowbuf[r, sl] * av
        pltpu.sync_copy(rowbuf, out_sh.at[d2d.at[ci]], add=True)
        return carry

    lax.fori_loop(0, _NCH, _chunk, 0)

    plsc.subcore_barrier()
    pltpu.sync_copy(out_sh.at[pl.ds(rbase, _RPT)],
                    out_hbm.at[c, pl.ds(rbase, _RPT)])


def _agg(h, s2d, d2d, ex2d):
    mesh = plsc.VectorSubcoreMesh(core_axis_name="c", subcore_axis_name="s")
    f = pl.kernel(
        _agg_body,
        out_type=jax.ShapeDtypeStruct((_NC, _NP, _D), jnp.float32),
        mesh=mesh,
        compiler_params=pltpu.CompilerParams(needs_layout_passes=False),
        scratch_types=[
            pltpu.VMEM((_NCH, _CH), jnp.int32),
            pltpu.VMEM((_NCH, _CH), jnp.int32),
            pltpu.VMEM((_NCH, _CH), jnp.float32),
            pltpu.VMEM((_CH, _D), jnp.float32),
            pltpu.VMEM_SHARED((_NP, _D), jnp.float32),
            pltpu.SemaphoreType.DMA,
        ],
    )
    return f(h, s2d, d2d, ex2d)


def _gat_sc(h, srcp, dstp, es, ed):
    ex, den_p = _edge_w(srcp, dstp, es, ed)
    out_p = _agg(h, srcp.reshape(_NW * _NCH, _CH),
                 dstp.reshape(_NW * _NCH, _CH),
                 ex.reshape(_NW * _NCH, _CH))
    return out_p, den_p.reshape(_NW, _N)


@jax.jit
def kernel(x, edge_index, batch, W1, a_src1, a_dst1, b1, W2, a_src2, a_dst2,
           b2, fcW1, fcb1, gamma, beta, fcW2, fcb2):
    srcp = jnp.pad(edge_index[0].reshape(_NW, _EPT),
                   ((0, 0), (0, _EPTP - _EPT))).reshape(_NW * _EPTP)
    dstp = jnp.pad(edge_index[1].reshape(_NW, _EPT),
                   ((0, 0), (0, _EPTP - _EPT))).reshape(_NW * _EPTP)
    batch3 = batch.reshape(_N // _BLK, 1, _BLK)

    h1, es1, ed1 = _proj(x, W1, a_src1, a_dst1)
    out_p1, den_p1 = _gat_sc(h1, srcp, dstp,
                             es1.reshape(_N), ed1.reshape(_N))
    h2, es2, ed2 = _merge_proj(out_p1[0, :_N], out_p1[1, :_N], den_p1.T, b1,
                               W2, a_src2, a_dst2)
    out_p2, den_p2 = _gat_sc(h2, srcp, dstp,
                             es2.reshape(_N), ed2.reshape(_N))
    return _final(out_p2[0, :_N], out_p2[1, :_N], den_p2.T, b2, batch3,
                  fcW1, fcb1, gamma, beta, fcW2, fcb2)
